# Initial kernel scaffold; baseline (speedup 1.0000x reference)
import jax
import jax.numpy as jnp
from jax.experimental import pallas as pl


def kernel(node_type, num_inverted_predecessors, edge_index, forward_level, backward_level, forward_index, backward_index, W_enc, b_enc, W_af, b_af, W_ab, b_ab, w_ih_f, w_hh_f, b_ih_f, b_hh_f, w_ih_b, w_hh_b, b_ih_b, b_hh_b):
    raise NotImplementedError("write your pallas kernel here")


# speedup vs baseline: 6.0257x; 6.0257x over previous
import functools

import jax
import jax.numpy as jnp
from jax import lax
from jax.experimental import pallas as pl
from jax.experimental.pallas import tpu as pltpu
from jax.experimental.pallas import tpu_sc as plsc

N = 10000
E = 320000
D = 128
H = 32
L = 8

NC = 2
NS = 16
NW = NC * NS
CHUNK = 128
CPW = 79
EPW = CPW * CHUNK
EPAD = EPW * NW
RPT = 632
P = RPT * NS
TRASH = N

BN = 1000
GRID = N // BN

@functools.cache
def _get_sc_levsel():
    mesh = plsc.VectorSubcoreMesh(core_axis_name="c", subcore_axis_name="s")

    @functools.partial(
        pl.kernel,
        mesh=mesh,
        compiler_params=pltpu.CompilerParams(use_tc_tiling_on_sc=False),
        out_type=jax.ShapeDtypeStruct((EPAD,), jnp.int32),
        scratch_types=[
            pltpu.VMEM((CHUNK,), jnp.int32),
            pltpu.VMEM((CHUNK,), jnp.int32),
            pltpu.SemaphoreType.DMA,
        ],
    )
    def _sc_levsel(sidx_hbm, lev_hbm, out_hbm, idxv, levg, sem):
        c = lax.axis_index("c")
        s = lax.axis_index("s")
        base = (s * NC + c) * EPW

        def body(i, carry):
            off = base + i * CHUNK
            pltpu.sync_copy(sidx_hbm.at[pl.ds(off, CHUNK)], idxv)
            pltpu.async_copy(lev_hbm.at[idxv], levg, sem).wait()
            pltpu.sync_copy(levg, out_hbm.at[pl.ds(off, CHUNK)])
            return carry

        lax.fori_loop(0, CPW, body, 0)

    return _sc_levsel


@functools.cache
def _get_sc_msg():
    mesh = plsc.VectorSubcoreMesh(core_axis_name="c", subcore_axis_name="s")

    @functools.partial(
        pl.kernel,
        mesh=mesh,
        compiler_params=pltpu.CompilerParams(use_tc_tiling_on_sc=False),
        out_type=jax.ShapeDtypeStruct((NC, P, H), jnp.float32),
        scratch_types=[
            pltpu.VMEM((CHUNK,), jnp.int32),
            pltpu.VMEM((CHUNK,), jnp.int32),
            pltpu.VMEM((CHUNK,), jnp.int32),
            pltpu.VMEM((CHUNK, H), jnp.float32),
            pltpu.VMEM((16,), jnp.int32),
            pltpu.VMEM_SHARED((P, H), jnp.float32),
            pltpu.SemaphoreType.DMA,
        ],
    )
    def _sc_msg(lvl16_hbm, gidx_hbm, sidx_hbm, slev_hbm, node_m_hbm, zeros_hbm,
                out_hbm, idx_g, idx_s, lvs, rows, lvlv, msg_sh, sem):
        c = lax.axis_index("c")
        s = lax.axis_index("s")
        wid = s * NC + c

        pltpu.sync_copy(lvl16_hbm, lvlv)
        pltpu.sync_copy(zeros_hbm, msg_sh.at[pl.ds(s * RPT, RPT)])
        plsc.subcore_barrier()

        base = wid * EPW
        lv = lvlv[...]

        def body(i, carry):
            off = base + i * CHUNK
            pltpu.sync_copy(gidx_hbm.at[pl.ds(off, CHUNK)], idx_g)
            pltpu.sync_copy(sidx_hbm.at[pl.ds(off, CHUNK)], idx_s)
            pltpu.sync_copy(slev_hbm.at[pl.ds(off, CHUNK)], lvs)
            for j in range(CHUNK // 16):
                dv = idx_s[pl.ds(j * 16, 16)]
                fv = lvs[pl.ds(j * 16, 16)]
                idx_s[pl.ds(j * 16, 16)] = jnp.where(fv == lv, dv, TRASH)
            pltpu.async_copy(node_m_hbm.at[idx_g], rows, sem).wait()
            pltpu.sync_copy(rows, msg_sh.at[idx_s], add=True)
            return carry

        lax.fori_loop(0, CPW, body, 0)

        plsc.subcore_barrier()
        pltpu.sync_copy(msg_sh.at[pl.ds(s * RPT, RPT)],
                        out_hbm.at[c, pl.ds(s * RPT, RPT)])

    return _sc_msg


def _init_body(x_ref, wenc_ref, benc_ref, waf_ref, baf_ref, h_ref, nm_ref):
    x = x_ref[...]
    h = jnp.dot(x, wenc_ref[...], preferred_element_type=jnp.float32) + benc_ref[...]
    h_ref[...] = h
    nm_ref[...] = jnp.dot(h, waf_ref[...], preferred_element_type=jnp.float32) + baf_ref[...]


def _gru_body(lvl_ref, h_ref, msg_ref, fl_ref, wih_ref, whh_ref, bih_ref,
              bhh_ref, wnx_ref, bnx_ref, h_out_ref, nm_ref):
    x = msg_ref[0] + msg_ref[1]
    hb = h_ref[...]
    dn = (((1,), (1,)), ((), ()))
    gi = lax.dot_general(x, wih_ref[...], dn, preferred_element_type=jnp.float32) + bih_ref[...]
    gh = lax.dot_general(hb, whh_ref[...], dn, preferred_element_type=jnp.float32) + bhh_ref[...]
    r = jax.nn.sigmoid(gi[:, :D] + gh[:, :D])
    z = jax.nn.sigmoid(gi[:, D:2 * D] + gh[:, D:2 * D])
    n = jnp.tanh(gi[:, 2 * D:] + r * gh[:, 2 * D:])
    new = (1.0 - z) * n + z * hb
    sel = fl_ref[...] == lvl_ref[0, 0]
    hn = jnp.where(sel, new, hb)
    h_out_ref[...] = hn
    nm_ref[...] = jnp.dot(hn, wnx_ref[...], preferred_element_type=jnp.float32) + bnx_ref[...]


def _full(shape):
    return pl.BlockSpec(shape, lambda i: tuple(0 for _ in shape))


_init_call = pl.pallas_call(
    _init_body,
    grid=(GRID,),
    in_specs=[
        pl.BlockSpec((BN, 2), lambda i: (i, 0)),
        _full((2, D)),
        _full((1, D)),
        _full((D, H)),
        _full((1, H)),
    ],
    out_specs=[
        pl.BlockSpec((BN, D), lambda i: (i, 0)),
        pl.BlockSpec((BN, H), lambda i: (i, 0)),
    ],
    out_shape=[
        jax.ShapeDtypeStruct((N, D), jnp.float32),
        jax.ShapeDtypeStruct((N, H), jnp.float32),
    ],
)

_gru_call = pl.pallas_call(
    _gru_body,
    grid=(GRID,),
    in_specs=[
        pl.BlockSpec(memory_space=pltpu.SMEM),
        pl.BlockSpec((BN, D), lambda i: (i, 0)),
        pl.BlockSpec((2, BN, H), lambda i: (0, i, 0)),
        pl.BlockSpec((BN, 1), lambda i: (i, 0)),
        _full((3 * D, H)),
        _full((3 * D, D)),
        _full((1, 3 * D)),
        _full((1, 3 * D)),
        _full((D, H)),
        _full((1, H)),
    ],
    out_specs=[
        pl.BlockSpec((BN, D), lambda i: (i, 0)),
        pl.BlockSpec((BN, H), lambda i: (i, 0)),
    ],
    out_shape=[
        jax.ShapeDtypeStruct((N, D), jnp.float32),
        jax.ShapeDtypeStruct((N, H), jnp.float32),
    ],
)


def kernel(node_type, num_inverted_predecessors, edge_index, forward_level,
           backward_level, forward_index, backward_index, W_enc, b_enc,
           W_af, b_af, W_ab, b_ab, w_ih_f, w_hh_f, b_ih_f, b_hh_f,
           w_ih_b, w_hh_b, b_ih_b, b_hh_b):
    f32 = jnp.float32
    xin = jnp.stack([node_type, num_inverted_predecessors], axis=1).astype(f32)
    h, nm = _init_call(xin, W_enc, b_enc.reshape(1, D), W_af, b_af.reshape(1, H))

    src = edge_index[0]
    dst = edge_index[1]
    pad_e = EPAD - E
    src_g = jnp.concatenate([src, jnp.zeros((pad_e,), jnp.int32)])
    dst_g = jnp.concatenate([dst, jnp.zeros((pad_e,), jnp.int32)])
    pad_t = jnp.full((pad_e,), TRASH, jnp.int32)
    src_s = jnp.concatenate([src, pad_t])
    dst_s = jnp.concatenate([dst, pad_t])
    flp = jnp.concatenate([forward_level, jnp.full((P - N,), -1, jnp.int32)])
    blp = jnp.concatenate([backward_level, jnp.full((P - N,), -1, jnp.int32)])
    fl2 = forward_level.reshape(N, 1)
    bl2 = backward_level.reshape(N, 1)
    zeros = jnp.zeros((RPT, H), f32)

    bih_f = b_ih_f.reshape(1, 3 * D)
    bhh_f = b_hh_f.reshape(1, 3 * D)
    bih_b = b_ih_b.reshape(1, 3 * D)
    bhh_b = b_hh_b.reshape(1, 3 * D)
    baf = b_af.reshape(1, H)
    bab = b_ab.reshape(1, H)

    slev_f = _get_sc_levsel()(dst_s, flp)
    slev_b = _get_sc_levsel()(src_s, blp)

    for l in range(1, L):
        lvl16 = jnp.full((16,), l, jnp.int32)
        msgp = _get_sc_msg()(lvl16, src_g, dst_s, slev_f, nm, zeros)
        wnx, bnx = (W_af, baf) if l < L - 1 else (W_ab, bab)
        h, nm = _gru_call(jnp.full((1, 1), l, jnp.int32), h,
                          msgp[:, :N, :], fl2, w_ih_f, w_hh_f, bih_f, bhh_f,
                          wnx, bnx)
    for l in range(1, L):
        lvl16 = jnp.full((16,), l, jnp.int32)
        msgp = _get_sc_msg()(lvl16, dst_g, src_s, slev_b, nm, zeros)
        h, nm = _gru_call(jnp.full((1, 1), l, jnp.int32), h,
                          msgp[:, :N, :], bl2, w_ih_b, w_hh_b, bih_b, bhh_b,
                          W_ab, bab)
    return h


# speedup vs baseline: 10.5363x; 1.7486x over previous
import functools

import jax
import jax.numpy as jnp
from jax import lax
from jax.experimental import pallas as pl
from jax.experimental.pallas import tpu as pltpu
from jax.experimental.pallas import tpu_sc as plsc

N = 10000
E = 320000
D = 128
H = 32
L = 8

NC = 2
NS = 16
NW = NC * NS
CHUNK = 128
CPW = 79
EPW = CPW * CHUNK
EPAD = EPW * NW
RPT = 632
P = RPT * NS
TRASH = N
RW = 8

BN = 1000
GRID = N // BN

@functools.cache
def _get_sc_levsel():
    mesh = plsc.VectorSubcoreMesh(core_axis_name="c", subcore_axis_name="s")

    @functools.partial(
        pl.kernel,
        mesh=mesh,
        compiler_params=pltpu.CompilerParams(use_tc_tiling_on_sc=False),
        out_type=jax.ShapeDtypeStruct((EPAD,), jnp.int32),
        scratch_types=[
            pltpu.VMEM((CHUNK,), jnp.int32),
            pltpu.VMEM((CHUNK,), jnp.int32),
            pltpu.SemaphoreType.DMA,
        ],
    )
    def _sc_levsel(sidx_hbm, lev_hbm, out_hbm, idxv, levg, sem):
        c = lax.axis_index("c")
        s = lax.axis_index("s")
        base = (s * NC + c) * EPW

        def body(i, carry):
            off = base + i * CHUNK
            pltpu.sync_copy(sidx_hbm.at[pl.ds(off, CHUNK)], idxv)
            pltpu.async_copy(lev_hbm.at[idxv], levg, sem).wait()
            pltpu.sync_copy(levg, out_hbm.at[pl.ds(off, CHUNK)])
            return carry

        lax.fori_loop(0, CPW, body, 0)

    return _sc_levsel


@functools.cache
def _get_sc_sort():
    mesh = plsc.VectorSubcoreMesh(core_axis_name="c", subcore_axis_name="s")

    @functools.partial(
        pl.kernel,
        mesh=mesh,
        compiler_params=pltpu.CompilerParams(use_tc_tiling_on_sc=False),
        out_type=jax.ShapeDtypeStruct((EPAD, RW), jnp.int32),
        scratch_types=[
            pltpu.VMEM((CHUNK,), jnp.int32),
            pltpu.VMEM((CHUNK, RW), jnp.int32),
            pltpu.SemaphoreType.DMA,
        ],
    )
    def _sc_sort(rec_hbm, dest_hbm, out_hbm, destv, recv, sem):
        c = lax.axis_index("c")
        s = lax.axis_index("s")
        base = (s * NC + c) * EPW

        def body(i, carry):
            off = base + i * CHUNK
            pltpu.sync_copy(dest_hbm.at[pl.ds(off, CHUNK)], destv)
            pltpu.sync_copy(rec_hbm.at[pl.ds(off, CHUNK)], recv)
            pltpu.async_copy(recv, out_hbm.at[destv], sem).wait()
            return carry

        lax.fori_loop(0, CPW, body, 0)

    return _sc_sort


@functools.cache
def _get_sc_msg():
    mesh = plsc.VectorSubcoreMesh(core_axis_name="c", subcore_axis_name="s")

    @functools.partial(
        pl.kernel,
        mesh=mesh,
        compiler_params=pltpu.CompilerParams(use_tc_tiling_on_sc=False),
        out_type=jax.ShapeDtypeStruct((NC, P, H), jnp.float32),
        scratch_types=[
            pltpu.VMEM((16,), jnp.int32),
            pltpu.VMEM((3 * CHUNK,), jnp.int32),
            pltpu.VMEM((CHUNK,), jnp.int32),
            pltpu.VMEM((CHUNK, H), jnp.float32),
            pltpu.VMEM((16,), jnp.int32),
            pltpu.VMEM_SHARED((P, H), jnp.float32),
            pltpu.SemaphoreType.DMA,
        ],
    )
    def _sc_msg(lvl16_hbm, bnd_hbm, meta_hbm,
                node_m_hbm, zeros_hbm, out_hbm, bsm, mv, idx_s, rows,
                lvlv, msg_sh, sem):
        c = lax.axis_index("c")
        s = lax.axis_index("s")
        wid = s * NC + c
        cbase = wid * CPW

        pltpu.sync_copy(lvl16_hbm, lvlv)
        pltpu.sync_copy(bnd_hbm.at[wid], bsm)
        pltpu.sync_copy(zeros_hbm, msg_sh.at[pl.ds(s * RPT, RPT)])
        plsc.subcore_barrier()

        lv = lvlv[...]
        bv = bsm[...]
        lo = bv[0]
        hi = bv[1]

        def body(i, carry):
            g = cbase + i
            pltpu.sync_copy(meta_hbm.at[pl.ds(g * (3 * CHUNK), 3 * CHUNK)], mv)
            for j in range(CHUNK // 16):
                dv = mv[pl.ds(CHUNK + j * 16, 16)]
                fvj = mv[pl.ds(2 * CHUNK + j * 16, 16)]
                idx_s[pl.ds(j * 16, 16)] = jnp.where(fvj == lv, dv, TRASH)
            pltpu.async_copy(node_m_hbm.at[mv.at[pl.ds(0, CHUNK)]],
                             rows, sem).wait()
            pltpu.sync_copy(rows, msg_sh.at[idx_s], add=True)
            return carry

        lax.fori_loop(lo, hi, body, 0)

        plsc.subcore_barrier()
        pltpu.sync_copy(msg_sh.at[pl.ds(s * RPT, RPT)],
                        out_hbm.at[c, pl.ds(s * RPT, RPT)])

    return _sc_msg


def _init_body(x_ref, wenc_ref, benc_ref, waf_ref, baf_ref, h_ref, nm_ref):
    x = x_ref[...]
    h = jnp.dot(x, wenc_ref[...], preferred_element_type=jnp.float32) + benc_ref[...]
    h_ref[...] = h
    nm_ref[...] = jnp.dot(h, waf_ref[...], preferred_element_type=jnp.float32) + baf_ref[...]


def _gru_body(lvl_ref, h_ref, msg_ref, fl_ref, wih_ref, whh_ref, bih_ref,
              bhh_ref, wnx_ref, bnx_ref, h_out_ref, nm_ref):
    x = msg_ref[0] + msg_ref[1]
    hb = h_ref[...]
    dn = (((1,), (1,)), ((), ()))
    gi = lax.dot_general(x, wih_ref[...], dn, preferred_element_type=jnp.float32) + bih_ref[...]
    gh = lax.dot_general(hb, whh_ref[...], dn, preferred_element_type=jnp.float32) + bhh_ref[...]
    r = jax.nn.sigmoid(gi[:, :D] + gh[:, :D])
    z = jax.nn.sigmoid(gi[:, D:2 * D] + gh[:, D:2 * D])
    n = jnp.tanh(gi[:, 2 * D:] + r * gh[:, 2 * D:])
    new = (1.0 - z) * n + z * hb
    sel = fl_ref[...] == lvl_ref[0, 0]
    hn = jnp.where(sel, new, hb)
    h_out_ref[...] = hn
    nm_ref[...] = jnp.dot(hn, wnx_ref[...], preferred_element_type=jnp.float32) + bnx_ref[...]


def _full(shape):
    return pl.BlockSpec(shape, lambda i: tuple(0 for _ in shape))


_init_call = pl.pallas_call(
    _init_body,
    grid=(GRID,),
    in_specs=[
        pl.BlockSpec((BN, 2), lambda i: (i, 0)),
        _full((2, D)),
        _full((1, D)),
        _full((D, H)),
        _full((1, H)),
    ],
    out_specs=[
        pl.BlockSpec((BN, D), lambda i: (i, 0)),
        pl.BlockSpec((BN, H), lambda i: (i, 0)),
    ],
    out_shape=[
        jax.ShapeDtypeStruct((N, D), jnp.float32),
        jax.ShapeDtypeStruct((N, H), jnp.float32),
    ],
)

_gru_call = pl.pallas_call(
    _gru_body,
    grid=(GRID,),
    in_specs=[
        pl.BlockSpec(memory_space=pltpu.SMEM),
        pl.BlockSpec((BN, D), lambda i: (i, 0)),
        pl.BlockSpec((2, BN, H), lambda i: (0, i, 0)),
        pl.BlockSpec((BN, 1), lambda i: (i, 0)),
        _full((3 * D, H)),
        _full((3 * D, D)),
        _full((1, 3 * D)),
        _full((1, 3 * D)),
        _full((D, H)),
        _full((1, H)),
    ],
    out_specs=[
        pl.BlockSpec((BN, D), lambda i: (i, 0)),
        pl.BlockSpec((BN, H), lambda i: (i, 0)),
    ],
    out_shape=[
        jax.ShapeDtypeStruct((N, D), jnp.float32),
        jax.ShapeDtypeStruct((N, H), jnp.float32),
    ],
)


def _level_sort(gidx, sidx, slev):
    key = jnp.where((slev >= 1) & (slev < L), slev, L) - 1
    oh = (key[:, None] == jnp.arange(L, dtype=jnp.int32)[None, :]).astype(jnp.int32)
    csum = jnp.cumsum(oh, axis=0)
    cnts = csum[-1]
    S = jnp.concatenate([jnp.zeros((1,), jnp.int32),
                         jnp.cumsum(cnts)]).astype(jnp.int32)
    rank = jnp.take_along_axis(csum, key[:, None], axis=1)[:, 0] - 1
    dest = S[key] + rank
    dest = (dest % NW) * EPW + dest // NW

    rec = jnp.stack([gidx, sidx, slev], axis=1)
    rec = jnp.pad(rec, ((0, 0), (0, RW - 3)))
    srec = _get_sc_sort()(rec, dest)

    meta = srec[:, :3].reshape(-1, CHUNK, 3).transpose(0, 2, 1).reshape(-1)

    w = jnp.arange(NW, dtype=jnp.int32)[None, :]
    lo_slot = (S[0:L - 1, None] + NW - 1 - w) // NW
    hi_slot = (S[1:L, None] + NW - 1 - w) // NW
    c_lo = lo_slot // CHUNK
    c_hi = (hi_slot + CHUNK - 1) // CHUNK
    bnd = jnp.zeros((L - 1, NW, 16), jnp.int32)
    bnd = bnd.at[:, :, 0].set(c_lo).at[:, :, 1].set(jnp.maximum(c_hi, c_lo))
    return meta, bnd


def kernel(node_type, num_inverted_predecessors, edge_index, forward_level,
           backward_level, forward_index, backward_index, W_enc, b_enc,
           W_af, b_af, W_ab, b_ab, w_ih_f, w_hh_f, b_ih_f, b_hh_f,
           w_ih_b, w_hh_b, b_ih_b, b_hh_b):
    f32 = jnp.float32
    xin = jnp.stack([node_type, num_inverted_predecessors], axis=1).astype(f32)
    h, nm = _init_call(xin, W_enc, b_enc.reshape(1, D), W_af, b_af.reshape(1, H))

    src = edge_index[0]
    dst = edge_index[1]
    pad_e = EPAD - E
    src_g = jnp.concatenate([src, jnp.zeros((pad_e,), jnp.int32)])
    dst_g = jnp.concatenate([dst, jnp.zeros((pad_e,), jnp.int32)])
    pad_t = jnp.full((pad_e,), TRASH, jnp.int32)
    src_s = jnp.concatenate([src, pad_t])
    dst_s = jnp.concatenate([dst, pad_t])
    flp = jnp.concatenate([forward_level, jnp.full((P - N,), -1, jnp.int32)])
    blp = jnp.concatenate([backward_level, jnp.full((P - N,), -1, jnp.int32)])
    fl2 = forward_level.reshape(N, 1)
    bl2 = backward_level.reshape(N, 1)
    zeros = jnp.zeros((RPT, H), f32)

    bih_f = b_ih_f.reshape(1, 3 * D)
    bhh_f = b_hh_f.reshape(1, 3 * D)
    bih_b = b_ih_b.reshape(1, 3 * D)
    bhh_b = b_hh_b.reshape(1, 3 * D)
    baf = b_af.reshape(1, H)
    bab = b_ab.reshape(1, H)

    slev_f = _get_sc_levsel()(dst_s, flp)
    slev_b = _get_sc_levsel()(src_s, blp)

    meta_f, bnd_f = _level_sort(src_g, dst_s, slev_f)
    meta_b, bnd_b = _level_sort(dst_g, src_s, slev_b)

    for l in range(1, L):
        lvl16 = jnp.full((16,), l, jnp.int32)
        msgp = _get_sc_msg()(lvl16, bnd_f[l - 1], meta_f, nm, zeros)
        wnx, bnx = (W_af, baf) if l < L - 1 else (W_ab, bab)
        h, nm = _gru_call(jnp.full((1, 1), l, jnp.int32), h,
                          msgp[:, :N, :], fl2, w_ih_f, w_hh_f, bih_f, bhh_f,
                          wnx, bnx)
    for l in range(1, L):
        lvl16 = jnp.full((16,), l, jnp.int32)
        msgp = _get_sc_msg()(lvl16, bnd_b[l - 1], meta_b, nm, zeros)
        h, nm = _gru_call(jnp.full((1, 1), l, jnp.int32), h,
                          msgp[:, :N, :], bl2, w_ih_b, w_hh_b, bih_b, bhh_b,
                          W_ab, bab)
    return h


# speedup vs baseline: 10.8310x; 1.0280x over previous
import functools

import jax
import jax.numpy as jnp
from jax import lax
from jax.experimental import pallas as pl
from jax.experimental.pallas import tpu as pltpu
from jax.experimental.pallas import tpu_sc as plsc

N = 10000
E = 320000
D = 128
H = 32
L = 8

NC = 2
NS = 16
NW = NC * NS
CHUNK = 128
CPW = 79
EPW = CPW * CHUNK
EPAD = EPW * NW
RPT = 632
P = RPT * NS
TRASH = N
RW = 8

BN = 1000
GRID = N // BN

@functools.cache
def _get_sc_levsel():
    mesh = plsc.VectorSubcoreMesh(core_axis_name="c", subcore_axis_name="s")

    @functools.partial(
        pl.kernel,
        mesh=mesh,
        compiler_params=pltpu.CompilerParams(use_tc_tiling_on_sc=False),
        out_type=jax.ShapeDtypeStruct((EPAD,), jnp.int32),
        scratch_types=[
            pltpu.VMEM((CHUNK,), jnp.int32),
            pltpu.VMEM((CHUNK,), jnp.int32),
            pltpu.SemaphoreType.DMA,
        ],
    )
    def _sc_levsel(sidx_hbm, lev_hbm, out_hbm, idxv, levg, sem):
        c = lax.axis_index("c")
        s = lax.axis_index("s")
        base = (s * NC + c) * EPW

        def body(i, carry):
            off = base + i * CHUNK
            pltpu.sync_copy(sidx_hbm.at[pl.ds(off, CHUNK)], idxv)
            pltpu.async_copy(lev_hbm.at[idxv], levg, sem).wait()
            pltpu.sync_copy(levg, out_hbm.at[pl.ds(off, CHUNK)])
            return carry

        lax.fori_loop(0, CPW, body, 0)

    return _sc_levsel


@functools.cache
def _get_sc_sort():
    mesh = plsc.VectorSubcoreMesh(core_axis_name="c", subcore_axis_name="s")

    @functools.partial(
        pl.kernel,
        mesh=mesh,
        compiler_params=pltpu.CompilerParams(use_tc_tiling_on_sc=False),
        out_type=jax.ShapeDtypeStruct((EPAD, RW), jnp.int32),
        scratch_types=[
            pltpu.VMEM((CHUNK,), jnp.int32),
            pltpu.VMEM((CHUNK, RW), jnp.int32),
            pltpu.SemaphoreType.DMA,
        ],
    )
    def _sc_sort(rec_hbm, dest_hbm, out_hbm, destv, recv, sem):
        c = lax.axis_index("c")
        s = lax.axis_index("s")
        base = (s * NC + c) * EPW

        def body(i, carry):
            off = base + i * CHUNK
            pltpu.sync_copy(dest_hbm.at[pl.ds(off, CHUNK)], destv)
            pltpu.sync_copy(rec_hbm.at[pl.ds(off, CHUNK)], recv)
            pltpu.async_copy(recv, out_hbm.at[destv], sem).wait()
            return carry

        lax.fori_loop(0, CPW, body, 0)

    return _sc_sort


@functools.cache
def _get_sc_msg():
    mesh = plsc.VectorSubcoreMesh(core_axis_name="c", subcore_axis_name="s")

    @functools.partial(
        pl.kernel,
        mesh=mesh,
        compiler_params=pltpu.CompilerParams(use_tc_tiling_on_sc=False),
        out_type=jax.ShapeDtypeStruct((NC, P, H), jnp.float32),
        scratch_types=[
            pltpu.VMEM((16,), jnp.int32),
            pltpu.VMEM((6 * CHUNK,), jnp.int32),
            pltpu.VMEM((CHUNK,), jnp.int32),
            pltpu.VMEM((CHUNK,), jnp.int32),
            pltpu.VMEM((CHUNK, H), jnp.float32),
            pltpu.VMEM((CHUNK, H), jnp.float32),
            pltpu.VMEM((16,), jnp.int32),
            pltpu.VMEM_SHARED((P, H), jnp.float32),
            pltpu.SemaphoreType.DMA,
            pltpu.SemaphoreType.DMA,
        ],
    )
    def _sc_msg(lvl16_hbm, bnd_hbm, meta_hbm,
                node_m_hbm, zeros_hbm, out_hbm, bsm, mv, idx_a, idx_b,
                rows_a, rows_b, lvlv, msg_sh, sem_a, sem_b):
        c = lax.axis_index("c")
        s = lax.axis_index("s")
        wid = s * NC + c
        cbase = wid * CPW

        pltpu.sync_copy(lvl16_hbm, lvlv)
        pltpu.sync_copy(bnd_hbm.at[wid], bsm)
        pltpu.sync_copy(zeros_hbm, msg_sh.at[pl.ds(s * RPT, RPT)])
        plsc.subcore_barrier()

        lv = lvlv[...]
        bv = bsm[...]
        lo = bv[0]
        hi = bv[1]
        npairs = (hi - lo) // 2

        def mask_idx(off, idx_ref):
            for j in range(CHUNK // 16):
                dv = mv[pl.ds(off + CHUNK + j * 16, 16)]
                fvj = mv[pl.ds(off + 2 * CHUNK + j * 16, 16)]
                idx_ref[pl.ds(j * 16, 16)] = jnp.where(fvj == lv, dv, TRASH)

        def pair_body(k, carry):
            g = cbase + lo + 2 * k
            pltpu.sync_copy(meta_hbm.at[pl.ds(g * (3 * CHUNK), 6 * CHUNK)], mv)
            mask_idx(0, idx_a)
            mask_idx(3 * CHUNK, idx_b)
            ga = pltpu.async_copy(node_m_hbm.at[mv.at[pl.ds(0, CHUNK)]],
                                  rows_a, sem_a)
            gb = pltpu.async_copy(node_m_hbm.at[mv.at[pl.ds(3 * CHUNK, CHUNK)]],
                                  rows_b, sem_b)
            ga.wait()
            pltpu.sync_copy(rows_a, msg_sh.at[idx_a], add=True)
            gb.wait()
            pltpu.sync_copy(rows_b, msg_sh.at[idx_b], add=True)
            return carry

        lax.fori_loop(0, npairs, pair_body, 0)

        def tail_body(i, carry):
            g = cbase + i
            pltpu.sync_copy(meta_hbm.at[pl.ds(g * (3 * CHUNK), 3 * CHUNK)], mv.at[pl.ds(0, 3 * CHUNK)])
            mask_idx(0, idx_a)
            pltpu.async_copy(node_m_hbm.at[mv.at[pl.ds(0, CHUNK)]],
                             rows_a, sem_a).wait()
            pltpu.sync_copy(rows_a, msg_sh.at[idx_a], add=True)
            return carry

        lax.fori_loop(lo + 2 * npairs, hi, tail_body, 0)

        plsc.subcore_barrier()
        pltpu.sync_copy(msg_sh.at[pl.ds(s * RPT, RPT)],
                        out_hbm.at[c, pl.ds(s * RPT, RPT)])

    return _sc_msg


def _init_body(x_ref, wenc_ref, benc_ref, waf_ref, baf_ref, h_ref, nm_ref):
    x = x_ref[...]
    h = jnp.dot(x, wenc_ref[...], preferred_element_type=jnp.float32) + benc_ref[...]
    h_ref[...] = h
    nm_ref[...] = jnp.dot(h, waf_ref[...], preferred_element_type=jnp.float32) + baf_ref[...]


def _gru_body(lvl_ref, h_ref, msg_ref, fl_ref, wih_ref, whh_ref, bih_ref,
              bhh_ref, wnx_ref, bnx_ref, h_out_ref, nm_ref):
    x = msg_ref[0] + msg_ref[1]
    hb = h_ref[...]
    dn = (((1,), (1,)), ((), ()))
    gi = lax.dot_general(x, wih_ref[...], dn, preferred_element_type=jnp.float32) + bih_ref[...]
    gh = lax.dot_general(hb, whh_ref[...], dn, preferred_element_type=jnp.float32) + bhh_ref[...]
    r = jax.nn.sigmoid(gi[:, :D] + gh[:, :D])
    z = jax.nn.sigmoid(gi[:, D:2 * D] + gh[:, D:2 * D])
    n = jnp.tanh(gi[:, 2 * D:] + r * gh[:, 2 * D:])
    new = (1.0 - z) * n + z * hb
    sel = fl_ref[...] == lvl_ref[0, 0]
    hn = jnp.where(sel, new, hb)
    h_out_ref[...] = hn
    nm_ref[...] = jnp.dot(hn, wnx_ref[...], preferred_element_type=jnp.float32) + bnx_ref[...]


def _full(shape):
    return pl.BlockSpec(shape, lambda i: tuple(0 for _ in shape))


_init_call = pl.pallas_call(
    _init_body,
    grid=(GRID,),
    in_specs=[
        pl.BlockSpec((BN, 2), lambda i: (i, 0)),
        _full((2, D)),
        _full((1, D)),
        _full((D, H)),
        _full((1, H)),
    ],
    out_specs=[
        pl.BlockSpec((BN, D), lambda i: (i, 0)),
        pl.BlockSpec((BN, H), lambda i: (i, 0)),
    ],
    out_shape=[
        jax.ShapeDtypeStruct((N, D), jnp.float32),
        jax.ShapeDtypeStruct((N, H), jnp.float32),
    ],
)

_gru_call = pl.pallas_call(
    _gru_body,
    grid=(GRID,),
    in_specs=[
        pl.BlockSpec(memory_space=pltpu.SMEM),
        pl.BlockSpec((BN, D), lambda i: (i, 0)),
        pl.BlockSpec((2, BN, H), lambda i: (0, i, 0)),
        pl.BlockSpec((BN, 1), lambda i: (i, 0)),
        _full((3 * D, H)),
        _full((3 * D, D)),
        _full((1, 3 * D)),
        _full((1, 3 * D)),
        _full((D, H)),
        _full((1, H)),
    ],
    out_specs=[
        pl.BlockSpec((BN, D), lambda i: (i, 0)),
        pl.BlockSpec((BN, H), lambda i: (i, 0)),
    ],
    out_shape=[
        jax.ShapeDtypeStruct((N, D), jnp.float32),
        jax.ShapeDtypeStruct((N, H), jnp.float32),
    ],
)


def _level_sort(gidx, sidx, slev):
    key = jnp.where((slev >= 1) & (slev < L), slev, L) - 1
    oh = (key[:, None] == jnp.arange(L, dtype=jnp.int32)[None, :]).astype(jnp.int32)
    csum = jnp.cumsum(oh, axis=0)
    cnts = csum[-1]
    S = jnp.concatenate([jnp.zeros((1,), jnp.int32),
                         jnp.cumsum(cnts)]).astype(jnp.int32)
    rank = jnp.take_along_axis(csum, key[:, None], axis=1)[:, 0] - 1
    dest = S[key] + rank
    dest = (dest % NW) * EPW + dest // NW

    rec = jnp.stack([gidx, sidx, slev], axis=1)
    rec = jnp.pad(rec, ((0, 0), (0, RW - 3)))
    srec = _get_sc_sort()(rec, dest)

    meta = srec[:, :3].reshape(-1, CHUNK, 3).transpose(0, 2, 1).reshape(-1)

    w = jnp.arange(NW, dtype=jnp.int32)[None, :]
    lo_slot = (S[0:L - 1, None] + NW - 1 - w) // NW
    hi_slot = (S[1:L, None] + NW - 1 - w) // NW
    c_lo = lo_slot // CHUNK
    c_hi = (hi_slot + CHUNK - 1) // CHUNK
    bnd = jnp.zeros((L - 1, NW, 16), jnp.int32)
    bnd = bnd.at[:, :, 0].set(c_lo).at[:, :, 1].set(jnp.maximum(c_hi, c_lo))
    return meta, bnd


def kernel(node_type, num_inverted_predecessors, edge_index, forward_level,
           backward_level, forward_index, backward_index, W_enc, b_enc,
           W_af, b_af, W_ab, b_ab, w_ih_f, w_hh_f, b_ih_f, b_hh_f,
           w_ih_b, w_hh_b, b_ih_b, b_hh_b):
    f32 = jnp.float32
    xin = jnp.stack([node_type, num_inverted_predecessors], axis=1).astype(f32)
    h, nm = _init_call(xin, W_enc, b_enc.reshape(1, D), W_af, b_af.reshape(1, H))

    src = edge_index[0]
    dst = edge_index[1]
    pad_e = EPAD - E
    src_g = jnp.concatenate([src, jnp.zeros((pad_e,), jnp.int32)])
    dst_g = jnp.concatenate([dst, jnp.zeros((pad_e,), jnp.int32)])
    pad_t = jnp.full((pad_e,), TRASH, jnp.int32)
    src_s = jnp.concatenate([src, pad_t])
    dst_s = jnp.concatenate([dst, pad_t])
    flp = jnp.concatenate([forward_level, jnp.full((P - N,), -1, jnp.int32)])
    blp = jnp.concatenate([backward_level, jnp.full((P - N,), -1, jnp.int32)])
    fl2 = forward_level.reshape(N, 1)
    bl2 = backward_level.reshape(N, 1)
    zeros = jnp.zeros((RPT, H), f32)

    bih_f = b_ih_f.reshape(1, 3 * D)
    bhh_f = b_hh_f.reshape(1, 3 * D)
    bih_b = b_ih_b.reshape(1, 3 * D)
    bhh_b = b_hh_b.reshape(1, 3 * D)
    baf = b_af.reshape(1, H)
    bab = b_ab.reshape(1, H)

    slev_f = _get_sc_levsel()(dst_s, flp)
    slev_b = _get_sc_levsel()(src_s, blp)

    meta_f, bnd_f = _level_sort(src_g, dst_s, slev_f)
    meta_b, bnd_b = _level_sort(dst_g, src_s, slev_b)

    for l in range(1, L):
        lvl16 = jnp.full((16,), l, jnp.int32)
        msgp = _get_sc_msg()(lvl16, bnd_f[l - 1], meta_f, nm, zeros)
        wnx, bnx = (W_af, baf) if l < L - 1 else (W_ab, bab)
        h, nm = _gru_call(jnp.full((1, 1), l, jnp.int32), h,
                          msgp[:, :N, :], fl2, w_ih_f, w_hh_f, bih_f, bhh_f,
                          wnx, bnx)
    for l in range(1, L):
        lvl16 = jnp.full((16,), l, jnp.int32)
        msgp = _get_sc_msg()(lvl16, bnd_b[l - 1], meta_b, nm, zeros)
        h, nm = _gru_call(jnp.full((1, 1), l, jnp.int32), h,
                          msgp[:, :N, :], bl2, w_ih_b, w_hh_b, bih_b, bhh_b,
                          W_ab, bab)
    return h


# speedup vs baseline: 11.9219x; 1.1007x over previous
import functools

import jax
import jax.numpy as jnp
from jax import lax
from jax.experimental import pallas as pl
from jax.experimental.pallas import tpu as pltpu
from jax.experimental.pallas import tpu_sc as plsc

N = 10000
E = 320000
D = 128
H = 32
L = 8

NC = 2
NS = 16
NW = NC * NS
CHUNK = 128
CPW = 79
EPW = CPW * CHUNK
EPAD = EPW * NW
RPT = 632
P = RPT * NS
TRASH = N
RW = 8

BN = 1000
GRID = N // BN

@functools.cache
def _get_sc_levsel():
    mesh = plsc.VectorSubcoreMesh(core_axis_name="c", subcore_axis_name="s")

    @functools.partial(
        pl.kernel,
        mesh=mesh,
        compiler_params=pltpu.CompilerParams(use_tc_tiling_on_sc=False),
        out_type=jax.ShapeDtypeStruct((EPAD,), jnp.int32),
        scratch_types=[
            pltpu.VMEM((2 * CHUNK,), jnp.int32),
            pltpu.VMEM((CHUNK,), jnp.int32),
            pltpu.VMEM((CHUNK,), jnp.int32),
            pltpu.SemaphoreType.DMA,
            pltpu.SemaphoreType.DMA,
            pltpu.SemaphoreType.DMA,
            pltpu.SemaphoreType.DMA,
        ],
    )
    def _sc_levsel(sidx_hbm, lev_hbm, out_hbm, idxv, lev_a, lev_b,
                   sem_a, sem_b, sem_wa, sem_wb):
        c = lax.axis_index("c")
        s = lax.axis_index("s")
        base = (s * NC + c) * EPW

        def pair_body(k, carry):
            off = base + 2 * k * CHUNK
            pltpu.sync_copy(sidx_hbm.at[pl.ds(off, 2 * CHUNK)], idxv)
            ga = pltpu.async_copy(lev_hbm.at[idxv.at[pl.ds(0, CHUNK)]],
                                  lev_a, sem_a)
            gb = pltpu.async_copy(lev_hbm.at[idxv.at[pl.ds(CHUNK, CHUNK)]],
                                  lev_b, sem_b)
            ga.wait()
            wa = pltpu.async_copy(lev_a, out_hbm.at[pl.ds(off, CHUNK)], sem_wa)
            gb.wait()
            wb = pltpu.async_copy(lev_b, out_hbm.at[pl.ds(off + CHUNK, CHUNK)],
                                  sem_wb)
            wa.wait()
            wb.wait()
            return carry

        lax.fori_loop(0, CPW // 2, pair_body, 0)

        def tail_body(i, carry):
            off = base + i * CHUNK
            pltpu.sync_copy(sidx_hbm.at[pl.ds(off, CHUNK)],
                            idxv.at[pl.ds(0, CHUNK)])
            pltpu.async_copy(lev_hbm.at[idxv.at[pl.ds(0, CHUNK)]],
                             lev_a, sem_a).wait()
            pltpu.sync_copy(lev_a, out_hbm.at[pl.ds(off, CHUNK)])
            return carry

        lax.fori_loop(2 * (CPW // 2), CPW, tail_body, 0)

    return _sc_levsel


@functools.cache
def _get_sc_sort():
    mesh = plsc.VectorSubcoreMesh(core_axis_name="c", subcore_axis_name="s")

    @functools.partial(
        pl.kernel,
        mesh=mesh,
        compiler_params=pltpu.CompilerParams(use_tc_tiling_on_sc=False),
        out_type=jax.ShapeDtypeStruct((EPAD, RW), jnp.int32),
        scratch_types=[
            pltpu.VMEM((CHUNK,), jnp.int32),
            pltpu.VMEM((CHUNK,), jnp.int32),
            pltpu.VMEM((2 * CHUNK, RW), jnp.int32),
            pltpu.SemaphoreType.DMA,
            pltpu.SemaphoreType.DMA,
            pltpu.SemaphoreType.DMA,
            pltpu.SemaphoreType.DMA,
            pltpu.SemaphoreType.DMA,
        ],
    )
    def _sc_sort(rec_hbm, dest_hbm, out_hbm, dest_a, dest_b, recv,
                 sem_da, sem_db, sem_r, sem_sa, sem_sb):
        c = lax.axis_index("c")
        s = lax.axis_index("s")
        base = (s * NC + c) * EPW

        def pair_body(k, carry):
            off = base + 2 * k * CHUNK
            la = pltpu.async_copy(dest_hbm.at[pl.ds(off, CHUNK)], dest_a, sem_da)
            lb = pltpu.async_copy(dest_hbm.at[pl.ds(off + CHUNK, CHUNK)],
                                  dest_b, sem_db)
            lr = pltpu.async_copy(rec_hbm.at[pl.ds(off, 2 * CHUNK)], recv, sem_r)
            la.wait()
            lr.wait()
            sa = pltpu.async_copy(recv.at[pl.ds(0, CHUNK)],
                                  out_hbm.at[dest_a], sem_sa)
            lb.wait()
            sb = pltpu.async_copy(recv.at[pl.ds(CHUNK, CHUNK)],
                                  out_hbm.at[dest_b], sem_sb)
            sa.wait()
            sb.wait()
            return carry

        lax.fori_loop(0, CPW // 2, pair_body, 0)

        def tail_body(i, carry):
            off = base + i * CHUNK
            pltpu.sync_copy(dest_hbm.at[pl.ds(off, CHUNK)], dest_a)
            pltpu.sync_copy(rec_hbm.at[pl.ds(off, CHUNK)],
                            recv.at[pl.ds(0, CHUNK)])
            pltpu.async_copy(recv.at[pl.ds(0, CHUNK)],
                             out_hbm.at[dest_a], sem_sa).wait()
            return carry

        lax.fori_loop(2 * (CPW // 2), CPW, tail_body, 0)

    return _sc_sort


@functools.cache
def _get_sc_msg():
    mesh = plsc.VectorSubcoreMesh(core_axis_name="c", subcore_axis_name="s")

    @functools.partial(
        pl.kernel,
        mesh=mesh,
        compiler_params=pltpu.CompilerParams(use_tc_tiling_on_sc=False),
        out_type=jax.ShapeDtypeStruct((NC, P, H), jnp.float32),
        scratch_types=[
            pltpu.VMEM((16,), jnp.int32),
            pltpu.VMEM((6 * CHUNK,), jnp.int32),
            pltpu.VMEM((CHUNK,), jnp.int32),
            pltpu.VMEM((CHUNK,), jnp.int32),
            pltpu.VMEM((CHUNK, H), jnp.float32),
            pltpu.VMEM((CHUNK, H), jnp.float32),
            pltpu.VMEM((16,), jnp.int32),
            pltpu.VMEM_SHARED((P, H), jnp.float32),
            pltpu.SemaphoreType.DMA,
            pltpu.SemaphoreType.DMA,
        ],
    )
    def _sc_msg(lvl16_hbm, bnd_hbm, meta_hbm,
                node_m_hbm, zeros_hbm, out_hbm, bsm, mv, idx_a, idx_b,
                rows_a, rows_b, lvlv, msg_sh, sem_a, sem_b):
        c = lax.axis_index("c")
        s = lax.axis_index("s")
        wid = s * NC + c
        cbase = wid * CPW

        pltpu.sync_copy(lvl16_hbm, lvlv)
        pltpu.sync_copy(bnd_hbm.at[wid], bsm)
        pltpu.sync_copy(zeros_hbm, msg_sh.at[pl.ds(s * RPT, RPT)])
        plsc.subcore_barrier()

        lv = lvlv[...]
        bv = bsm[...]
        lo = bv[0]
        hi = bv[1]
        npairs = (hi - lo) // 2

        def mask_idx(off, idx_ref):
            for j in range(CHUNK // 16):
                dv = mv[pl.ds(off + CHUNK + j * 16, 16)]
                fvj = mv[pl.ds(off + 2 * CHUNK + j * 16, 16)]
                idx_ref[pl.ds(j * 16, 16)] = jnp.where(fvj == lv, dv, TRASH)

        def pair_body(k, carry):
            g = cbase + lo + 2 * k
            pltpu.sync_copy(meta_hbm.at[pl.ds(g * (3 * CHUNK), 6 * CHUNK)], mv)
            mask_idx(0, idx_a)
            mask_idx(3 * CHUNK, idx_b)
            ga = pltpu.async_copy(node_m_hbm.at[mv.at[pl.ds(0, CHUNK)]],
                                  rows_a, sem_a)
            gb = pltpu.async_copy(node_m_hbm.at[mv.at[pl.ds(3 * CHUNK, CHUNK)]],
                                  rows_b, sem_b)
            ga.wait()
            pltpu.sync_copy(rows_a, msg_sh.at[idx_a], add=True)
            gb.wait()
            pltpu.sync_copy(rows_b, msg_sh.at[idx_b], add=True)
            return carry

        lax.fori_loop(0, npairs, pair_body, 0)

        def tail_body(i, carry):
            g = cbase + i
            pltpu.sync_copy(meta_hbm.at[pl.ds(g * (3 * CHUNK), 3 * CHUNK)], mv.at[pl.ds(0, 3 * CHUNK)])
            mask_idx(0, idx_a)
            pltpu.async_copy(node_m_hbm.at[mv.at[pl.ds(0, CHUNK)]],
                             rows_a, sem_a).wait()
            pltpu.sync_copy(rows_a, msg_sh.at[idx_a], add=True)
            return carry

        lax.fori_loop(lo + 2 * npairs, hi, tail_body, 0)

        plsc.subcore_barrier()
        pltpu.sync_copy(msg_sh.at[pl.ds(s * RPT, RPT)],
                        out_hbm.at[c, pl.ds(s * RPT, RPT)])

    return _sc_msg


def _init_body(x_ref, wenc_ref, benc_ref, waf_ref, baf_ref, h_ref, nm_ref):
    x = x_ref[...]
    h = jnp.dot(x, wenc_ref[...], preferred_element_type=jnp.float32) + benc_ref[...]
    h_ref[...] = h
    nm_ref[...] = jnp.dot(h, waf_ref[...], preferred_element_type=jnp.float32) + baf_ref[...]


def _gru_body(lvl_ref, h_ref, msg_ref, fl_ref, wih_ref, whh_ref, bih_ref,
              bhh_ref, wnx_ref, bnx_ref, h_out_ref, nm_ref):
    x = msg_ref[0] + msg_ref[1]
    hb = h_ref[...]
    dn = (((1,), (1,)), ((), ()))
    gi = lax.dot_general(x, wih_ref[...], dn, preferred_element_type=jnp.float32) + bih_ref[...]
    gh = lax.dot_general(hb, whh_ref[...], dn, preferred_element_type=jnp.float32) + bhh_ref[...]
    r = jax.nn.sigmoid(gi[:, :D] + gh[:, :D])
    z = jax.nn.sigmoid(gi[:, D:2 * D] + gh[:, D:2 * D])
    n = jnp.tanh(gi[:, 2 * D:] + r * gh[:, 2 * D:])
    new = (1.0 - z) * n + z * hb
    sel = fl_ref[...] == lvl_ref[0, 0]
    hn = jnp.where(sel, new, hb)
    h_out_ref[...] = hn
    nm_ref[...] = jnp.dot(hn, wnx_ref[...], preferred_element_type=jnp.float32) + bnx_ref[...]


def _full(shape):
    return pl.BlockSpec(shape, lambda i: tuple(0 for _ in shape))


_init_call = pl.pallas_call(
    _init_body,
    grid=(GRID,),
    in_specs=[
        pl.BlockSpec((BN, 2), lambda i: (i, 0)),
        _full((2, D)),
        _full((1, D)),
        _full((D, H)),
        _full((1, H)),
    ],
    out_specs=[
        pl.BlockSpec((BN, D), lambda i: (i, 0)),
        pl.BlockSpec((BN, H), lambda i: (i, 0)),
    ],
    out_shape=[
        jax.ShapeDtypeStruct((N, D), jnp.float32),
        jax.ShapeDtypeStruct((N, H), jnp.float32),
    ],
)

_gru_call = pl.pallas_call(
    _gru_body,
    grid=(GRID,),
    in_specs=[
        pl.BlockSpec(memory_space=pltpu.SMEM),
        pl.BlockSpec((BN, D), lambda i: (i, 0)),
        pl.BlockSpec((2, BN, H), lambda i: (0, i, 0)),
        pl.BlockSpec((BN, 1), lambda i: (i, 0)),
        _full((3 * D, H)),
        _full((3 * D, D)),
        _full((1, 3 * D)),
        _full((1, 3 * D)),
        _full((D, H)),
        _full((1, H)),
    ],
    out_specs=[
        pl.BlockSpec((BN, D), lambda i: (i, 0)),
        pl.BlockSpec((BN, H), lambda i: (i, 0)),
    ],
    out_shape=[
        jax.ShapeDtypeStruct((N, D), jnp.float32),
        jax.ShapeDtypeStruct((N, H), jnp.float32),
    ],
)


def _level_sort(gidx, sidx, slev):
    key = jnp.where((slev >= 1) & (slev < L), slev, L) - 1
    oh = (key[:, None] == jnp.arange(L, dtype=jnp.int32)[None, :]).astype(jnp.int32)
    csum = jnp.cumsum(oh, axis=0)
    cnts = csum[-1]
    S = jnp.concatenate([jnp.zeros((1,), jnp.int32),
                         jnp.cumsum(cnts)]).astype(jnp.int32)
    rank = jnp.take_along_axis(csum, key[:, None], axis=1)[:, 0] - 1
    dest = S[key] + rank
    dest = (dest % NW) * EPW + dest // NW

    rec = jnp.stack([gidx, sidx, slev], axis=1)
    rec = jnp.pad(rec, ((0, 0), (0, RW - 3)))
    srec = _get_sc_sort()(rec, dest)

    meta = srec[:, :3].reshape(-1, CHUNK, 3).transpose(0, 2, 1).reshape(-1)

    w = jnp.arange(NW, dtype=jnp.int32)[None, :]
    lo_slot = (S[0:L - 1, None] + NW - 1 - w) // NW
    hi_slot = (S[1:L, None] + NW - 1 - w) // NW
    c_lo = lo_slot // CHUNK
    c_hi = (hi_slot + CHUNK - 1) // CHUNK
    bnd = jnp.zeros((L - 1, NW, 16), jnp.int32)
    bnd = bnd.at[:, :, 0].set(c_lo).at[:, :, 1].set(jnp.maximum(c_hi, c_lo))
    return meta, bnd


def kernel(node_type, num_inverted_predecessors, edge_index, forward_level,
           backward_level, forward_index, backward_index, W_enc, b_enc,
           W_af, b_af, W_ab, b_ab, w_ih_f, w_hh_f, b_ih_f, b_hh_f,
           w_ih_b, w_hh_b, b_ih_b, b_hh_b):
    f32 = jnp.float32
    xin = jnp.stack([node_type, num_inverted_predecessors], axis=1).astype(f32)
    h, nm = _init_call(xin, W_enc, b_enc.reshape(1, D), W_af, b_af.reshape(1, H))

    src = edge_index[0]
    dst = edge_index[1]
    pad_e = EPAD - E
    src_g = jnp.concatenate([src, jnp.zeros((pad_e,), jnp.int32)])
    dst_g = jnp.concatenate([dst, jnp.zeros((pad_e,), jnp.int32)])
    pad_t = jnp.full((pad_e,), TRASH, jnp.int32)
    src_s = jnp.concatenate([src, pad_t])
    dst_s = jnp.concatenate([dst, pad_t])
    flp = jnp.concatenate([forward_level, jnp.full((P - N,), -1, jnp.int32)])
    blp = jnp.concatenate([backward_level, jnp.full((P - N,), -1, jnp.int32)])
    fl2 = forward_level.reshape(N, 1)
    bl2 = backward_level.reshape(N, 1)
    zeros = jnp.zeros((RPT, H), f32)

    bih_f = b_ih_f.reshape(1, 3 * D)
    bhh_f = b_hh_f.reshape(1, 3 * D)
    bih_b = b_ih_b.reshape(1, 3 * D)
    bhh_b = b_hh_b.reshape(1, 3 * D)
    baf = b_af.reshape(1, H)
    bab = b_ab.reshape(1, H)

    slev_f = _get_sc_levsel()(dst_s, flp)
    slev_b = _get_sc_levsel()(src_s, blp)

    meta_f, bnd_f = _level_sort(src_g, dst_s, slev_f)
    meta_b, bnd_b = _level_sort(dst_g, src_s, slev_b)

    for l in range(1, L):
        lvl16 = jnp.full((16,), l, jnp.int32)
        msgp = _get_sc_msg()(lvl16, bnd_f[l - 1], meta_f, nm, zeros)
        wnx, bnx = (W_af, baf) if l < L - 1 else (W_ab, bab)
        h, nm = _gru_call(jnp.full((1, 1), l, jnp.int32), h,
                          msgp[:, :N, :], fl2, w_ih_f, w_hh_f, bih_f, bhh_f,
                          wnx, bnx)
    for l in range(1, L):
        lvl16 = jnp.full((16,), l, jnp.int32)
        msgp = _get_sc_msg()(lvl16, bnd_b[l - 1], meta_b, nm, zeros)
        h, nm = _gru_call(jnp.full((1, 1), l, jnp.int32), h,
                          msgp[:, :N, :], bl2, w_ih_b, w_hh_b, bih_b, bhh_b,
                          W_ab, bab)
    return h


# speedup vs baseline: 12.1117x; 1.0159x over previous
import functools

import jax
import jax.numpy as jnp
from jax import lax
from jax.experimental import pallas as pl
from jax.experimental.pallas import tpu as pltpu
from jax.experimental.pallas import tpu_sc as plsc

N = 10000
E = 320000
D = 128
H = 32
L = 8

NC = 2
NS = 16
NW = NC * NS
CHUNK = 128
CPW = 79
EPW = CPW * CHUNK
EPAD = EPW * NW
RPT = 632
P = RPT * NS
TRASH = N
RW = 8

BN = 1000
GRID = N // BN

@functools.cache
def _get_sc_levsel():
    mesh = plsc.VectorSubcoreMesh(core_axis_name="c", subcore_axis_name="s")

    @functools.partial(
        pl.kernel,
        mesh=mesh,
        compiler_params=pltpu.CompilerParams(use_tc_tiling_on_sc=False),
        out_type=jax.ShapeDtypeStruct((2 * EPAD,), jnp.int32),
        scratch_types=[
            pltpu.VMEM((2 * CHUNK,), jnp.int32),
            pltpu.VMEM((CHUNK,), jnp.int32),
            pltpu.VMEM((CHUNK,), jnp.int32),
            pltpu.SemaphoreType.DMA,
            pltpu.SemaphoreType.DMA,
            pltpu.SemaphoreType.DMA,
            pltpu.SemaphoreType.DMA,
        ],
    )
    def _sc_levsel(sidx_hbm, lev_hbm, out_hbm, idxv, lev_a, lev_b,
                   sem_a, sem_b, sem_wa, sem_wb):
        c = lax.axis_index("c")
        s = lax.axis_index("s")
        base = (s * NC + c) * (2 * EPW)

        def pair_body(k, carry):
            off = base + 2 * k * CHUNK
            pltpu.sync_copy(sidx_hbm.at[pl.ds(off, 2 * CHUNK)], idxv)
            ga = pltpu.async_copy(lev_hbm.at[idxv.at[pl.ds(0, CHUNK)]],
                                  lev_a, sem_a)
            gb = pltpu.async_copy(lev_hbm.at[idxv.at[pl.ds(CHUNK, CHUNK)]],
                                  lev_b, sem_b)
            ga.wait()
            wa = pltpu.async_copy(lev_a, out_hbm.at[pl.ds(off, CHUNK)], sem_wa)
            gb.wait()
            wb = pltpu.async_copy(lev_b, out_hbm.at[pl.ds(off + CHUNK, CHUNK)],
                                  sem_wb)
            wa.wait()
            wb.wait()
            return carry

        lax.fori_loop(0, CPW, pair_body, 0)

    return _sc_levsel


@functools.cache
def _get_sc_sort():
    mesh = plsc.VectorSubcoreMesh(core_axis_name="c", subcore_axis_name="s")

    @functools.partial(
        pl.kernel,
        mesh=mesh,
        compiler_params=pltpu.CompilerParams(use_tc_tiling_on_sc=False),
        out_type=jax.ShapeDtypeStruct((2 * EPAD, RW), jnp.int32),
        scratch_types=[
            pltpu.VMEM((CHUNK,), jnp.int32),
            pltpu.VMEM((CHUNK,), jnp.int32),
            pltpu.VMEM((2 * CHUNK, RW), jnp.int32),
            pltpu.SemaphoreType.DMA,
            pltpu.SemaphoreType.DMA,
            pltpu.SemaphoreType.DMA,
            pltpu.SemaphoreType.DMA,
            pltpu.SemaphoreType.DMA,
        ],
    )
    def _sc_sort(rec_hbm, dest_hbm, out_hbm, dest_a, dest_b, recv,
                 sem_da, sem_db, sem_r, sem_sa, sem_sb):
        c = lax.axis_index("c")
        s = lax.axis_index("s")
        base = (s * NC + c) * (2 * EPW)

        def pair_body(k, carry):
            off = base + 2 * k * CHUNK
            la = pltpu.async_copy(dest_hbm.at[pl.ds(off, CHUNK)], dest_a, sem_da)
            lb = pltpu.async_copy(dest_hbm.at[pl.ds(off + CHUNK, CHUNK)],
                                  dest_b, sem_db)
            lr = pltpu.async_copy(rec_hbm.at[pl.ds(off, 2 * CHUNK)], recv, sem_r)
            la.wait()
            lr.wait()
            sa = pltpu.async_copy(recv.at[pl.ds(0, CHUNK)],
                                  out_hbm.at[dest_a], sem_sa)
            lb.wait()
            sb = pltpu.async_copy(recv.at[pl.ds(CHUNK, CHUNK)],
                                  out_hbm.at[dest_b], sem_sb)
            sa.wait()
            sb.wait()
            return carry

        lax.fori_loop(0, CPW, pair_body, 0)

    return _sc_sort


@functools.cache
def _get_sc_msg():
    mesh = plsc.VectorSubcoreMesh(core_axis_name="c", subcore_axis_name="s")

    @functools.partial(
        pl.kernel,
        mesh=mesh,
        compiler_params=pltpu.CompilerParams(use_tc_tiling_on_sc=False),
        out_type=jax.ShapeDtypeStruct((NC, P, H), jnp.float32),
        scratch_types=[
            pltpu.VMEM((16,), jnp.int32),
            pltpu.VMEM((6 * CHUNK,), jnp.int32),
            pltpu.VMEM((CHUNK,), jnp.int32),
            pltpu.VMEM((CHUNK,), jnp.int32),
            pltpu.VMEM((CHUNK, H), jnp.float32),
            pltpu.VMEM((CHUNK, H), jnp.float32),
            pltpu.VMEM((16,), jnp.int32),
            pltpu.VMEM_SHARED((P, H), jnp.float32),
            pltpu.SemaphoreType.DMA,
            pltpu.SemaphoreType.DMA,
        ],
    )
    def _sc_msg(lvl16_hbm, bnd_hbm, meta_hbm,
                node_m_hbm, zeros_hbm, out_hbm, bsm, mv, idx_a, idx_b,
                rows_a, rows_b, lvlv, msg_sh, sem_a, sem_b):
        c = lax.axis_index("c")
        s = lax.axis_index("s")
        wid = s * NC + c
        cbase = wid * CPW

        pltpu.sync_copy(lvl16_hbm, lvlv)
        pltpu.sync_copy(bnd_hbm.at[wid], bsm)
        pltpu.sync_copy(zeros_hbm, msg_sh.at[pl.ds(s * RPT, RPT)])
        plsc.subcore_barrier()

        lv = lvlv[...]
        bv = bsm[...]
        lo = bv[0]
        hi = bv[1]
        npairs = (hi - lo) // 2

        def mask_idx(off, idx_ref):
            for j in range(CHUNK // 16):
                dv = mv[pl.ds(off + CHUNK + j * 16, 16)]
                fvj = mv[pl.ds(off + 2 * CHUNK + j * 16, 16)]
                idx_ref[pl.ds(j * 16, 16)] = jnp.where(fvj == lv, dv, TRASH)

        def pair_body(k, carry):
            g = cbase + lo + 2 * k
            pltpu.sync_copy(meta_hbm.at[pl.ds(g * (3 * CHUNK), 6 * CHUNK)], mv)
            mask_idx(0, idx_a)
            mask_idx(3 * CHUNK, idx_b)
            ga = pltpu.async_copy(node_m_hbm.at[mv.at[pl.ds(0, CHUNK)]],
                                  rows_a, sem_a)
            gb = pltpu.async_copy(node_m_hbm.at[mv.at[pl.ds(3 * CHUNK, CHUNK)]],
                                  rows_b, sem_b)
            ga.wait()
            pltpu.sync_copy(rows_a, msg_sh.at[idx_a], add=True)
            gb.wait()
            pltpu.sync_copy(rows_b, msg_sh.at[idx_b], add=True)
            return carry

        lax.fori_loop(0, npairs, pair_body, 0)

        def tail_body(i, carry):
            g = cbase + i
            pltpu.sync_copy(meta_hbm.at[pl.ds(g * (3 * CHUNK), 3 * CHUNK)], mv.at[pl.ds(0, 3 * CHUNK)])
            mask_idx(0, idx_a)
            pltpu.async_copy(node_m_hbm.at[mv.at[pl.ds(0, CHUNK)]],
                             rows_a, sem_a).wait()
            pltpu.sync_copy(rows_a, msg_sh.at[idx_a], add=True)
            return carry

        lax.fori_loop(lo + 2 * npairs, hi, tail_body, 0)

        plsc.subcore_barrier()
        pltpu.sync_copy(msg_sh.at[pl.ds(s * RPT, RPT)],
                        out_hbm.at[c, pl.ds(s * RPT, RPT)])

    return _sc_msg


def _init_body(x_ref, wenc_ref, benc_ref, waf_ref, baf_ref, h_ref, nm_ref):
    x = x_ref[...]
    h = jnp.dot(x, wenc_ref[...], preferred_element_type=jnp.float32) + benc_ref[...]
    h_ref[...] = h
    nm_ref[...] = jnp.dot(h, waf_ref[...], preferred_element_type=jnp.float32) + baf_ref[...]


def _gru_body(lvl_ref, h_ref, msg_ref, fl_ref, wih_ref, whh_ref, bih_ref,
              bhh_ref, wnx_ref, bnx_ref, h_out_ref, nm_ref):
    x = msg_ref[0] + msg_ref[1]
    hb = h_ref[...]
    dn = (((1,), (1,)), ((), ()))
    gi = lax.dot_general(x, wih_ref[...], dn, preferred_element_type=jnp.float32) + bih_ref[...]
    gh = lax.dot_general(hb, whh_ref[...], dn, preferred_element_type=jnp.float32) + bhh_ref[...]
    r = jax.nn.sigmoid(gi[:, :D] + gh[:, :D])
    z = jax.nn.sigmoid(gi[:, D:2 * D] + gh[:, D:2 * D])
    n = jnp.tanh(gi[:, 2 * D:] + r * gh[:, 2 * D:])
    new = (1.0 - z) * n + z * hb
    sel = fl_ref[...] == lvl_ref[0, 0]
    hn = jnp.where(sel, new, hb)
    h_out_ref[...] = hn
    nm_ref[...] = jnp.dot(hn, wnx_ref[...], preferred_element_type=jnp.float32) + bnx_ref[...]


def _full(shape):
    return pl.BlockSpec(shape, lambda i: tuple(0 for _ in shape))


_init_call = pl.pallas_call(
    _init_body,
    grid=(GRID,),
    in_specs=[
        pl.BlockSpec((BN, 2), lambda i: (i, 0)),
        _full((2, D)),
        _full((1, D)),
        _full((D, H)),
        _full((1, H)),
    ],
    out_specs=[
        pl.BlockSpec((BN, D), lambda i: (i, 0)),
        pl.BlockSpec((BN, H), lambda i: (i, 0)),
    ],
    out_shape=[
        jax.ShapeDtypeStruct((N, D), jnp.float32),
        jax.ShapeDtypeStruct((N, H), jnp.float32),
    ],
)

_gru_call = pl.pallas_call(
    _gru_body,
    grid=(GRID,),
    in_specs=[
        pl.BlockSpec(memory_space=pltpu.SMEM),
        pl.BlockSpec((BN, D), lambda i: (i, 0)),
        pl.BlockSpec((2, BN, H), lambda i: (0, i, 0)),
        pl.BlockSpec((BN, 1), lambda i: (i, 0)),
        _full((3 * D, H)),
        _full((3 * D, D)),
        _full((1, 3 * D)),
        _full((1, 3 * D)),
        _full((D, H)),
        _full((1, H)),
    ],
    out_specs=[
        pl.BlockSpec((BN, D), lambda i: (i, 0)),
        pl.BlockSpec((BN, H), lambda i: (i, 0)),
    ],
    out_shape=[
        jax.ShapeDtypeStruct((N, D), jnp.float32),
        jax.ShapeDtypeStruct((N, H), jnp.float32),
    ],
)


def _sort_dest(gidx, sidx, slev):
    key = jnp.where((slev >= 1) & (slev < L), slev, L) - 1
    oh = (key[:, None] == jnp.arange(L, dtype=jnp.int32)[None, :]).astype(jnp.int32)
    csum = jnp.cumsum(oh, axis=0)
    cnts = csum[-1]
    S = jnp.concatenate([jnp.zeros((1,), jnp.int32),
                         jnp.cumsum(cnts)]).astype(jnp.int32)
    rank = jnp.take_along_axis(csum, key[:, None], axis=1)[:, 0] - 1
    dest = S[key] + rank
    dest = (dest % NW) * EPW + dest // NW
    rec = jnp.stack([gidx, sidx, slev], axis=1)
    rec = jnp.pad(rec, ((0, 0), (0, RW - 3)))
    return rec, dest, S


def _meta_bnd(srec, S):
    meta = srec[:, :3].reshape(-1, CHUNK, 3).transpose(0, 2, 1).reshape(-1)
    w = jnp.arange(NW, dtype=jnp.int32)[None, :]
    lo_slot = (S[0:L - 1, None] + NW - 1 - w) // NW
    hi_slot = (S[1:L, None] + NW - 1 - w) // NW
    c_lo = lo_slot // CHUNK
    c_hi = (hi_slot + CHUNK - 1) // CHUNK
    bnd = jnp.zeros((L - 1, NW, 16), jnp.int32)
    bnd = bnd.at[:, :, 0].set(c_lo).at[:, :, 1].set(jnp.maximum(c_hi, c_lo))
    return meta, bnd


def kernel(node_type, num_inverted_predecessors, edge_index, forward_level,
           backward_level, forward_index, backward_index, W_enc, b_enc,
           W_af, b_af, W_ab, b_ab, w_ih_f, w_hh_f, b_ih_f, b_hh_f,
           w_ih_b, w_hh_b, b_ih_b, b_hh_b):
    f32 = jnp.float32
    xin = jnp.stack([node_type, num_inverted_predecessors], axis=1).astype(f32)
    h, nm = _init_call(xin, W_enc, b_enc.reshape(1, D), W_af, b_af.reshape(1, H))

    src = edge_index[0]
    dst = edge_index[1]
    pad_e = EPAD - E
    src_g = jnp.concatenate([src, jnp.zeros((pad_e,), jnp.int32)])
    dst_g = jnp.concatenate([dst, jnp.zeros((pad_e,), jnp.int32)])
    pad_t = jnp.full((pad_e,), TRASH, jnp.int32)
    src_s = jnp.concatenate([src, pad_t])
    dst_s = jnp.concatenate([dst, pad_t])
    flp = jnp.concatenate([forward_level, jnp.full((P - N,), -1, jnp.int32)])
    blp = jnp.concatenate([backward_level, jnp.full((P - N,), -1, jnp.int32)])
    fl2 = forward_level.reshape(N, 1)
    bl2 = backward_level.reshape(N, 1)
    zeros = jnp.zeros((RPT, H), f32)

    bih_f = b_ih_f.reshape(1, 3 * D)
    bhh_f = b_hh_f.reshape(1, 3 * D)
    bih_b = b_ih_b.reshape(1, 3 * D)
    bhh_b = b_hh_b.reshape(1, 3 * D)
    baf = b_af.reshape(1, H)
    bab = b_ab.reshape(1, H)

    slev2 = _get_sc_levsel()(jnp.concatenate([dst_s, src_s + P]),
                             jnp.concatenate([flp, blp]))
    rec_f, dest_f, S_f = _sort_dest(src_g, dst_s, slev2[:EPAD])
    rec_b, dest_b, S_b = _sort_dest(dst_g, src_s, slev2[EPAD:])
    srec2 = _get_sc_sort()(jnp.concatenate([rec_f, rec_b], axis=0),
                           jnp.concatenate([dest_f, dest_b + EPAD]))
    meta_f, bnd_f = _meta_bnd(srec2[:EPAD], S_f)
    meta_b, bnd_b = _meta_bnd(srec2[EPAD:], S_b)

    for l in range(1, L):
        lvl16 = jnp.full((16,), l, jnp.int32)
        msgp = _get_sc_msg()(lvl16, bnd_f[l - 1], meta_f, nm, zeros)
        wnx, bnx = (W_af, baf) if l < L - 1 else (W_ab, bab)
        h, nm = _gru_call(jnp.full((1, 1), l, jnp.int32), h,
                          msgp[:, :N, :], fl2, w_ih_f, w_hh_f, bih_f, bhh_f,
                          wnx, bnx)
    for l in range(1, L):
        lvl16 = jnp.full((16,), l, jnp.int32)
        msgp = _get_sc_msg()(lvl16, bnd_b[l - 1], meta_b, nm, zeros)
        h, nm = _gru_call(jnp.full((1, 1), l, jnp.int32), h,
                          msgp[:, :N, :], bl2, w_ih_b, w_hh_b, bih_b, bhh_b,
                          W_ab, bab)
    return h
